# Initial kernel scaffold; baseline (speedup 1.0000x reference)
import jax
import jax.numpy as jnp
from jax.experimental import pallas as pl


def kernel(inputs, indexes, features, labels):
    raise NotImplementedError("write your pallas kernel here")


# speedup vs baseline: 11.6956x; 11.6956x over previous
import functools

import jax
import jax.numpy as jnp
from jax import lax
from jax.experimental import pallas as pl
from jax.experimental.pallas import tpu as pltpu
from jax.experimental.pallas import tpu_sc as plsc

N = 100000
F = 128
C = 1000
B = 1024
TEMP = 0.05

NC = 2
NS = 16
NW = NC * NS

BLK = 128
NFULL = N // BLK
TAIL = N - NFULL * BLK
TAIL_OFF = NFULL * BLK
NB_LO = NFULL // NW
NB_EXTRA = NFULL - NB_LO * NW
ZROWS = 64
ZLAST = C - 15 * ZROWS
BT = B // NW

@functools.cache
def _sc_segment_fn():
    mesh = plsc.VectorSubcoreMesh(
        core_axis_name="c", subcore_axis_name="s", num_cores=NC, num_subcores=NS
    )
    return functools.partial(
        pl.kernel,
        out_type=(
            jax.ShapeDtypeStruct((NC, C, F), jnp.float32),
            jax.ShapeDtypeStruct((NC, C, 16), jnp.float32),
            jax.ShapeDtypeStruct((B,), jnp.int32),
        ),
        mesh=mesh,
        scratch_types=(
            pltpu.VMEM((BLK,), jnp.int32),
            pltpu.VMEM((BLK, F), jnp.float32),
            pltpu.VMEM((BLK, 16), jnp.float32),
            pltpu.VMEM((ZROWS, F), jnp.float32),
            pltpu.VMEM((ZROWS, 16), jnp.float32),
            pltpu.VMEM((TAIL,), jnp.int32),
            pltpu.VMEM((TAIL, F), jnp.float32),
            pltpu.VMEM((BT,), jnp.int32),
            pltpu.VMEM((BT,), jnp.int32),
            pltpu.VMEM_SHARED((C, F), jnp.float32),
            pltpu.VMEM_SHARED((C, 16), jnp.float32),
            pltpu.SemaphoreType.DMA,
        ),
    )(_sc_body)


def _sc_body(feat_hbm, lab_hbm, idx_hbm, sums_out, cnts_out, tgt_out,
             lbl_v, rows_v, ones_v, zrow_v, zcnt_v, ltail_v, rtail_v,
             idx_v, tgt_v, acc_sh, cnt_sh, sem):
    cid = lax.axis_index("c")
    sid = lax.axis_index("s")
    wid = sid * NC + cid

    def _fill_ones(i, carry):
        ones_v[i, :] = jnp.ones((16,), jnp.float32)
        return carry

    lax.fori_loop(0, BLK, _fill_ones, 0)

    def _fill_zero(i, carry):
        for cchunk in range(F // 16):
            zrow_v[i, pl.ds(cchunk * 16, 16)] = jnp.zeros((16,), jnp.float32)
        zcnt_v[i, :] = jnp.zeros((16,), jnp.float32)
        return carry

    lax.fori_loop(0, ZROWS, _fill_zero, 0)

    tbase = pl.multiple_of(wid * BT, BT)
    pltpu.sync_copy(idx_hbm.at[pl.ds(tbase, BT)], idx_v)
    pltpu.async_copy(lab_hbm.at[idx_v], tgt_v, sem).wait()
    pltpu.sync_copy(tgt_v, tgt_out.at[pl.ds(tbase, BT)])

    zoff = sid * ZROWS

    @pl.when(sid < NS - 1)
    def _zero_main():
        pltpu.sync_copy(zrow_v, acc_sh.at[pl.ds(zoff, ZROWS)])
        pltpu.sync_copy(zcnt_v, cnt_sh.at[pl.ds(zoff, ZROWS)])

    @pl.when(sid == NS - 1)
    def _zero_last():
        pltpu.sync_copy(zrow_v.at[pl.ds(0, ZLAST)], acc_sh.at[pl.ds(15 * ZROWS, ZLAST)])
        pltpu.sync_copy(zcnt_v.at[pl.ds(0, ZLAST)], cnt_sh.at[pl.ds(15 * ZROWS, ZLAST)])

    plsc.subcore_barrier()

    nb = jnp.where(wid < NB_EXTRA, NB_LO + 1, NB_LO)

    def _block(k, carry):
        off = pl.multiple_of((wid + NW * k) * BLK, BLK)
        pltpu.sync_copy(lab_hbm.at[pl.ds(off, BLK)], lbl_v)
        pltpu.sync_copy(feat_hbm.at[pl.ds(off, BLK), :], rows_v)
        pltpu.sync_copy(rows_v, acc_sh.at[lbl_v], add=True)
        pltpu.sync_copy(ones_v, cnt_sh.at[lbl_v], add=True)
        return carry

    lax.fori_loop(0, nb, _block, 0)

    @pl.when(wid == NW - 1)
    def _tail():
        pltpu.sync_copy(lab_hbm.at[pl.ds(TAIL_OFF, TAIL)], ltail_v)
        pltpu.sync_copy(feat_hbm.at[pl.ds(TAIL_OFF, TAIL), :], rtail_v)
        pltpu.sync_copy(rtail_v, acc_sh.at[ltail_v], add=True)
        pltpu.sync_copy(ones_v.at[pl.ds(0, TAIL)], cnt_sh.at[ltail_v], add=True)

    plsc.subcore_barrier()

    @pl.when(sid < NS - 1)
    def _dump_main():
        pltpu.sync_copy(acc_sh.at[pl.ds(zoff, ZROWS)], sums_out.at[cid, pl.ds(zoff, ZROWS), :])
        pltpu.sync_copy(cnt_sh.at[pl.ds(zoff, ZROWS)], cnts_out.at[cid, pl.ds(zoff, ZROWS), :])

    @pl.when(sid == NS - 1)
    def _dump_last():
        pltpu.sync_copy(acc_sh.at[pl.ds(15 * ZROWS, ZLAST)],
                        sums_out.at[cid, pl.ds(15 * ZROWS, ZLAST), :])
        pltpu.sync_copy(cnt_sh.at[pl.ds(15 * ZROWS, ZLAST)],
                        cnts_out.at[cid, pl.ds(15 * ZROWS, ZLAST), :])


def _tc_body(x_ref, sums_ref, cnts_ref, tgt_ref, out_ref):
    x = x_ref[...]
    nrm = jnp.sqrt(jnp.sum(x * x, axis=1, keepdims=True))
    x = x / jnp.maximum(nrm, 1e-12)
    cs = sums_ref[0] + sums_ref[1]
    cnt16 = cnts_ref[0] + cnts_ref[1]
    cnt = jnp.max(cnt16, axis=1, keepdims=True)
    csd = cs / (jnp.maximum(cnt, 1.0) * TEMP)
    scores = lax.dot_general(x, csd, (((1,), (1,)), ((), ())),
                             preferred_element_type=jnp.float32)
    exps = jnp.exp(scores)
    rowsum = jnp.sum(exps, axis=1, keepdims=True)
    nmask = jnp.sum(jnp.where(cnt == 0.0, 1.0, 0.0))
    msum = rowsum - nmask + 1e-6
    tgt = tgt_ref[...]
    onehot = lax.broadcasted_iota(jnp.int32, (B, C), 1) == tgt
    s_t = jnp.sum(jnp.where(onehot, scores, 0.0), axis=1, keepdims=True)
    picked = jnp.log(jnp.exp(s_t) / msum + 1e-6)
    out_ref[0, 0] = -jnp.sum(picked) / jnp.float32(B)


_tc_loss = pl.pallas_call(
    _tc_body,
    out_shape=jax.ShapeDtypeStruct((1, 1), jnp.float32),
    out_specs=pl.BlockSpec(memory_space=pltpu.MemorySpace.SMEM),
)


def kernel(inputs, indexes, features, labels):
    sums, cnts, tgt = _sc_segment_fn()(features, labels, indexes)
    out = _tc_loss(inputs, sums, cnts, tgt.reshape(B, 1))
    return out[0, 0]


# speedup vs baseline: 18.3859x; 1.5720x over previous
import functools

import jax
import jax.numpy as jnp
from jax import lax
from jax.experimental import pallas as pl
from jax.experimental.pallas import tpu as pltpu
from jax.experimental.pallas import tpu_sc as plsc

N = 100000
F = 128
C = 1000
B = 1024
TEMP = 0.05

NC = 2
NS = 16
NW = NC * NS

BLK = 128
NFULL = N // BLK
TAIL = N - NFULL * BLK
TAIL_OFF = NFULL * BLK
NB_LO = NFULL // NW
NB_EXTRA = NFULL - NB_LO * NW
ZROWS = 64
ZLAST = C - 15 * ZROWS
BT = B // NW

@functools.cache
def _sc_segment_fn():
    mesh = plsc.VectorSubcoreMesh(
        core_axis_name="c", subcore_axis_name="s", num_cores=NC, num_subcores=NS
    )
    return functools.partial(
        pl.kernel,
        out_type=(
            jax.ShapeDtypeStruct((NC, C, F), jnp.float32),
            jax.ShapeDtypeStruct((NC, C, 16), jnp.float32),
            jax.ShapeDtypeStruct((B,), jnp.int32),
        ),
        mesh=mesh,
        scratch_types=(
            pltpu.VMEM((2, BLK), jnp.int32),
            pltpu.VMEM((2, BLK, F), jnp.float32),
            pltpu.VMEM((BLK, 16), jnp.float32),
            pltpu.VMEM((ZROWS, F), jnp.float32),
            pltpu.VMEM((ZROWS, 16), jnp.float32),
            pltpu.VMEM((TAIL,), jnp.int32),
            pltpu.VMEM((TAIL, F), jnp.float32),
            pltpu.VMEM((BT,), jnp.int32),
            pltpu.VMEM((BT,), jnp.int32),
            pltpu.VMEM_SHARED((C, F), jnp.float32),
            pltpu.VMEM_SHARED((C, 16), jnp.float32),
            pltpu.SemaphoreType.DMA((2,)),
            pltpu.SemaphoreType.DMA,
        ),
    )(_sc_body)


def _sc_body(feat_hbm, lab_hbm, idx_hbm, sums_out, cnts_out, tgt_out,
             lbl2, rows2, ones_v, zrow_v, zcnt_v, ltail_v, rtail_v,
             idx_v, tgt_v, acc_sh, cnt_sh, sems, sem):
    cid = lax.axis_index("c")
    sid = lax.axis_index("s")
    wid = sid * NC + cid

    def _fill_ones(i, carry):
        ones_v[i, :] = jnp.ones((16,), jnp.float32)
        return carry

    lax.fori_loop(0, BLK, _fill_ones, 0)

    def _fill_zero(i, carry):
        for cchunk in range(F // 16):
            zrow_v[i, pl.ds(cchunk * 16, 16)] = jnp.zeros((16,), jnp.float32)
        zcnt_v[i, :] = jnp.zeros((16,), jnp.float32)
        return carry

    lax.fori_loop(0, ZROWS, _fill_zero, 0)

    nb = jnp.where(wid < NB_EXTRA, NB_LO + 1, NB_LO)

    def _issue(k, slot):
        off = pl.multiple_of((wid + NW * k) * BLK, BLK)
        pltpu.async_copy(lab_hbm.at[pl.ds(off, BLK)], lbl2.at[slot], sems.at[slot])
        pltpu.async_copy(feat_hbm.at[pl.ds(off, BLK), :], rows2.at[slot], sems.at[slot])

    _issue(0, 0)
    _issue(1, 1)

    tbase = pl.multiple_of(wid * BT, BT)
    pltpu.sync_copy(idx_hbm.at[pl.ds(tbase, BT)], idx_v)
    pltpu.async_copy(lab_hbm.at[idx_v], tgt_v, sem).wait()
    pltpu.sync_copy(tgt_v, tgt_out.at[pl.ds(tbase, BT)])

    zoff = sid * ZROWS

    @pl.when(sid < NS - 1)
    def _zero_main():
        pltpu.sync_copy(zrow_v, acc_sh.at[pl.ds(zoff, ZROWS)])
        pltpu.sync_copy(zcnt_v, cnt_sh.at[pl.ds(zoff, ZROWS)])

    @pl.when(sid == NS - 1)
    def _zero_last():
        pltpu.sync_copy(zrow_v.at[pl.ds(0, ZLAST)], acc_sh.at[pl.ds(15 * ZROWS, ZLAST)])
        pltpu.sync_copy(zcnt_v.at[pl.ds(0, ZLAST)], cnt_sh.at[pl.ds(15 * ZROWS, ZLAST)])

    plsc.subcore_barrier()

    def _block(k, carry):
        slot = lax.rem(k, 2)
        off = pl.multiple_of((wid + NW * k) * BLK, BLK)
        pltpu.make_async_copy(lab_hbm.at[pl.ds(off, BLK)], lbl2.at[slot],
                              sems.at[slot]).wait()
        pltpu.make_async_copy(feat_hbm.at[pl.ds(off, BLK), :], rows2.at[slot],
                              sems.at[slot]).wait()
        pltpu.sync_copy(rows2.at[slot], acc_sh.at[lbl2.at[slot]], add=True)
        pltpu.sync_copy(ones_v, cnt_sh.at[lbl2.at[slot]], add=True)

        @pl.when(k + 2 < nb)
        def _next():
            _issue(k + 2, slot)

        return carry

    lax.fori_loop(0, nb, _block, 0)

    @pl.when(wid == NW - 1)
    def _tail():
        pltpu.sync_copy(lab_hbm.at[pl.ds(TAIL_OFF, TAIL)], ltail_v)
        pltpu.sync_copy(feat_hbm.at[pl.ds(TAIL_OFF, TAIL), :], rtail_v)
        pltpu.sync_copy(rtail_v, acc_sh.at[ltail_v], add=True)
        pltpu.sync_copy(ones_v.at[pl.ds(0, TAIL)], cnt_sh.at[ltail_v], add=True)

    plsc.subcore_barrier()

    @pl.when(sid < NS - 1)
    def _dump_main():
        pltpu.sync_copy(acc_sh.at[pl.ds(zoff, ZROWS)], sums_out.at[cid, pl.ds(zoff, ZROWS), :])
        pltpu.sync_copy(cnt_sh.at[pl.ds(zoff, ZROWS)], cnts_out.at[cid, pl.ds(zoff, ZROWS), :])

    @pl.when(sid == NS - 1)
    def _dump_last():
        pltpu.sync_copy(acc_sh.at[pl.ds(15 * ZROWS, ZLAST)],
                        sums_out.at[cid, pl.ds(15 * ZROWS, ZLAST), :])
        pltpu.sync_copy(cnt_sh.at[pl.ds(15 * ZROWS, ZLAST)],
                        cnts_out.at[cid, pl.ds(15 * ZROWS, ZLAST), :])


def _tc_body(x_ref, sums_ref, cnts_ref, tgt_ref, out_ref):
    x = x_ref[...]
    nrm = jnp.sqrt(jnp.sum(x * x, axis=1, keepdims=True))
    x = x / jnp.maximum(nrm, 1e-12)
    cs = sums_ref[0] + sums_ref[1]
    cnt16 = cnts_ref[0] + cnts_ref[1]
    cnt = jnp.max(cnt16, axis=1, keepdims=True)
    csd = cs / (jnp.maximum(cnt, 1.0) * TEMP)
    scores = lax.dot_general(x, csd, (((1,), (1,)), ((), ())),
                             preferred_element_type=jnp.float32)
    exps = jnp.exp(scores)
    rowsum = jnp.sum(exps, axis=1, keepdims=True)
    nmask = jnp.sum(jnp.where(cnt == 0.0, 1.0, 0.0))
    msum = rowsum - nmask + 1e-6
    tgt = tgt_ref[...]
    onehot = lax.broadcasted_iota(jnp.int32, (B, C), 1) == tgt
    s_t = jnp.sum(jnp.where(onehot, scores, 0.0), axis=1, keepdims=True)
    picked = jnp.log(jnp.exp(s_t) / msum + 1e-6)
    out_ref[0, 0] = -jnp.sum(picked) / jnp.float32(B)


_tc_loss = pl.pallas_call(
    _tc_body,
    out_shape=jax.ShapeDtypeStruct((1, 1), jnp.float32),
    out_specs=pl.BlockSpec(memory_space=pltpu.MemorySpace.SMEM),
)


def kernel(inputs, indexes, features, labels):
    sums, cnts, tgt = _sc_segment_fn()(features, labels, indexes)
    out = _tc_loss(inputs, sums, cnts, tgt.reshape(B, 1))
    return out[0, 0]


# speedup vs baseline: 18.4200x; 1.0019x over previous
import functools

import jax
import jax.numpy as jnp
from jax import lax
from jax.experimental import pallas as pl
from jax.experimental.pallas import tpu as pltpu
from jax.experimental.pallas import tpu_sc as plsc

N = 100000
F = 128
C = 1000
B = 1024
TEMP = 0.05

NC = 2
NS = 16
NW = NC * NS

BLK = 128
NFULL = N // BLK
TAIL = N - NFULL * BLK
TAIL_OFF = NFULL * BLK
NB_LO = NFULL // NW
NB_EXTRA = NFULL - NB_LO * NW
ZROWS = 64
ZLAST = C - 15 * ZROWS
BT = B // NW

@functools.cache
def _sc_segment_fn():
    mesh = plsc.VectorSubcoreMesh(
        core_axis_name="c", subcore_axis_name="s", num_cores=NC, num_subcores=NS
    )
    return functools.partial(
        pl.kernel,
        out_type=(
            jax.ShapeDtypeStruct((NC, C, F), jnp.float32),
            jax.ShapeDtypeStruct((NC, C, 16), jnp.float32),
            jax.ShapeDtypeStruct((B,), jnp.int32),
        ),
        mesh=mesh,
        scratch_types=(
            pltpu.VMEM((BLK,), jnp.int32),
            pltpu.VMEM((BLK,), jnp.int32),
            pltpu.VMEM((BLK, F), jnp.float32),
            pltpu.VMEM((BLK, F), jnp.float32),
            pltpu.VMEM((BLK, 16), jnp.float32),
            pltpu.VMEM((ZROWS, F), jnp.float32),
            pltpu.VMEM((ZROWS, 16), jnp.float32),
            pltpu.VMEM((TAIL,), jnp.int32),
            pltpu.VMEM((TAIL, F), jnp.float32),
            pltpu.VMEM((BT,), jnp.int32),
            pltpu.VMEM((BT,), jnp.int32),
            pltpu.VMEM_SHARED((C, F), jnp.float32),
            pltpu.VMEM_SHARED((C, 16), jnp.float32),
            pltpu.SemaphoreType.DMA,
            pltpu.SemaphoreType.DMA,
            pltpu.SemaphoreType.DMA,
        ),
    )(_sc_body)


def _sc_body(feat_hbm, lab_hbm, idx_hbm, sums_out, cnts_out, tgt_out,
             lblA, lblB, rowsA, rowsB, ones_v, zrow_v, zcnt_v, ltail_v, rtail_v,
             idx_v, tgt_v, acc_sh, cnt_sh, semA, semB, sem):
    cid = lax.axis_index("c")
    sid = lax.axis_index("s")
    wid = sid * NC + cid

    def _fill_ones(i, carry):
        ones_v[i, :] = jnp.ones((16,), jnp.float32)
        return carry

    lax.fori_loop(0, BLK, _fill_ones, 0)

    def _fill_zero(i, carry):
        for cchunk in range(F // 16):
            zrow_v[i, pl.ds(cchunk * 16, 16)] = jnp.zeros((16,), jnp.float32)
        zcnt_v[i, :] = jnp.zeros((16,), jnp.float32)
        return carry

    lax.fori_loop(0, ZROWS, _fill_zero, 0)

    nb = jnp.where(wid < NB_EXTRA, NB_LO + 1, NB_LO)
    slot_refs = ((lblA, rowsA, semA), (lblB, rowsB, semB))

    def _issue(k, slot):
        lbl_v, rows_v, sem_s = slot_refs[slot]
        off = pl.multiple_of((wid + NW * k) * BLK, BLK)
        pltpu.async_copy(lab_hbm.at[pl.ds(off, BLK)], lbl_v, sem_s)
        pltpu.async_copy(feat_hbm.at[pl.ds(off, BLK), :], rows_v, sem_s)

    _issue(0, 0)
    _issue(1, 1)

    tbase = pl.multiple_of(wid * BT, BT)
    pltpu.sync_copy(idx_hbm.at[pl.ds(tbase, BT)], idx_v)
    pltpu.async_copy(lab_hbm.at[idx_v], tgt_v, sem).wait()
    pltpu.sync_copy(tgt_v, tgt_out.at[pl.ds(tbase, BT)])

    zoff = sid * ZROWS

    @pl.when(sid < NS - 1)
    def _zero_main():
        pltpu.sync_copy(zrow_v, acc_sh.at[pl.ds(zoff, ZROWS)])
        pltpu.sync_copy(zcnt_v, cnt_sh.at[pl.ds(zoff, ZROWS)])

    @pl.when(sid == NS - 1)
    def _zero_last():
        pltpu.sync_copy(zrow_v.at[pl.ds(0, ZLAST)], acc_sh.at[pl.ds(15 * ZROWS, ZLAST)])
        pltpu.sync_copy(zcnt_v.at[pl.ds(0, ZLAST)], cnt_sh.at[pl.ds(15 * ZROWS, ZLAST)])

    plsc.subcore_barrier()

    def _pair(j, carry):
        for slot in range(2):
            k = 2 * j + slot
            lbl_v, rows_v, sem_s = slot_refs[slot]

            @pl.when(k < nb)
            def _do():
                off = pl.multiple_of((wid + NW * k) * BLK, BLK)
                pltpu.make_async_copy(lab_hbm.at[pl.ds(off, BLK)], lbl_v,
                                      sem_s).wait()
                pltpu.make_async_copy(feat_hbm.at[pl.ds(off, BLK), :],
                                      rows_v, sem_s).wait()
                pltpu.sync_copy(rows_v, acc_sh.at[lbl_v], add=True)
                pltpu.sync_copy(ones_v, cnt_sh.at[lbl_v], add=True)

                @pl.when(k + 2 < nb)
                def _next():
                    _issue(k + 2, slot)

        return carry

    lax.fori_loop(0, (NB_LO + 2) // 2, _pair, 0)

    @pl.when(wid == NW - 1)
    def _tail():
        pltpu.sync_copy(lab_hbm.at[pl.ds(TAIL_OFF, TAIL)], ltail_v)
        pltpu.sync_copy(feat_hbm.at[pl.ds(TAIL_OFF, TAIL), :], rtail_v)
        pltpu.sync_copy(rtail_v, acc_sh.at[ltail_v], add=True)
        pltpu.sync_copy(ones_v.at[pl.ds(0, TAIL)], cnt_sh.at[ltail_v], add=True)

    plsc.subcore_barrier()

    @pl.when(sid < NS - 1)
    def _dump_main():
        pltpu.sync_copy(acc_sh.at[pl.ds(zoff, ZROWS)], sums_out.at[cid, pl.ds(zoff, ZROWS), :])
        pltpu.sync_copy(cnt_sh.at[pl.ds(zoff, ZROWS)], cnts_out.at[cid, pl.ds(zoff, ZROWS), :])

    @pl.when(sid == NS - 1)
    def _dump_last():
        pltpu.sync_copy(acc_sh.at[pl.ds(15 * ZROWS, ZLAST)],
                        sums_out.at[cid, pl.ds(15 * ZROWS, ZLAST), :])
        pltpu.sync_copy(cnt_sh.at[pl.ds(15 * ZROWS, ZLAST)],
                        cnts_out.at[cid, pl.ds(15 * ZROWS, ZLAST), :])


def _tc_body(x_ref, sums_ref, cnts_ref, tgt_ref, out_ref):
    x = x_ref[...]
    nrm = jnp.sqrt(jnp.sum(x * x, axis=1, keepdims=True))
    x = x / jnp.maximum(nrm, 1e-12)
    cs = sums_ref[0] + sums_ref[1]
    cnt16 = cnts_ref[0] + cnts_ref[1]
    cnt = jnp.max(cnt16, axis=1, keepdims=True)
    csd = cs / (jnp.maximum(cnt, 1.0) * TEMP)
    scores = lax.dot_general(x, csd, (((1,), (1,)), ((), ())),
                             preferred_element_type=jnp.float32)
    exps = jnp.exp(scores)
    rowsum = jnp.sum(exps, axis=1, keepdims=True)
    nmask = jnp.sum(jnp.where(cnt == 0.0, 1.0, 0.0))
    msum = rowsum - nmask + 1e-6
    tgt = tgt_ref[...]
    onehot = lax.broadcasted_iota(jnp.int32, (B, C), 1) == tgt
    s_t = jnp.sum(jnp.where(onehot, scores, 0.0), axis=1, keepdims=True)
    picked = jnp.log(jnp.exp(s_t) / msum + 1e-6)
    out_ref[0, 0] = -jnp.sum(picked) / jnp.float32(B)


_tc_loss = pl.pallas_call(
    _tc_body,
    out_shape=jax.ShapeDtypeStruct((1, 1), jnp.float32),
    out_specs=pl.BlockSpec(memory_space=pltpu.MemorySpace.SMEM),
)


def kernel(inputs, indexes, features, labels):
    sums, cnts, tgt = _sc_segment_fn()(features, labels, indexes)
    out = _tc_loss(inputs, sums, cnts, tgt.reshape(B, 1))
    return out[0, 0]
